# baseline (speedup 1.0000x reference)
import functools

import jax
import jax.numpy as jnp
from jax.experimental import pallas as pl
from jax.experimental.pallas import tpu as pltpu
from jax.experimental.pallas import tpu_sc as plsc

T = 2048
HIDDEN = 1024
N_EXPERTS = 8
TOP_K = 2
D_FF = 768
D_FF_SHARED = 1536

TB = 256
L = T * TOP_K + N_EXPERTS * TB
NB = L // TB
NC = 2
NS = 16
NW = NC * NS
TPW = T // NW


def _dot_t(a, b):
    return jax.lax.dot_general(a.astype(jnp.bfloat16), b.astype(jnp.bfloat16),
                               (((1,), (1,)), ((), ())),
                               preferred_element_type=jnp.float32)


def _shift_down(a, s):
    return jnp.concatenate([jnp.zeros((s, a.shape[1]), a.dtype), a[:-s]],
                           axis=0)


def _dispatch_kernel(x_ref, gw_ref, bias_ref,
                     dest_ref, w1_ref, w2_ref, be_ref, used_ref):
    x = x_ref[...]
    logits = jax.lax.dot_general(x, gw_ref[...], (((1,), (1,)), ((), ())),
                                 preferred_element_type=jnp.float32)
    scores = jax.nn.sigmoid(logits)
    biased = scores + bias_ref[...]
    eiota = jax.lax.broadcasted_iota(jnp.int32, (T, N_EXPERTS), 1)
    m1 = jnp.max(biased, axis=1, keepdims=True)
    idx1 = jnp.min(jnp.where(biased == m1, eiota, N_EXPERTS),
                   axis=1, keepdims=True)
    oh1 = eiota == idx1
    b2 = jnp.where(oh1, -jnp.inf, biased)
    m2 = jnp.max(b2, axis=1, keepdims=True)
    idx2 = jnp.min(jnp.where(b2 == m2, eiota, N_EXPERTS),
                   axis=1, keepdims=True)
    oh2 = eiota == idx2
    w1 = jnp.sum(jnp.where(oh1, scores, 0.0), axis=1, keepdims=True)
    w2 = jnp.sum(jnp.where(oh2, scores, 0.0), axis=1, keepdims=True)
    den = w1 + w2 + 1e-20
    w1_ref[...] = jnp.broadcast_to(w1 / den, (T, 16))
    w2_ref[...] = jnp.broadcast_to(w2 / den, (T, 16))

    count = (oh1 | oh2).astype(jnp.float32)
    c = count
    s = 1
    while s < T:
        c = c + _shift_down(c, s)
        s *= 2
    c_exc = c - count
    cnt_row = c[T - 1:T, :]
    nblk_row = jnp.floor((cnt_row + (TB - 1)) / TB)
    used_ref[...] = jnp.sum(nblk_row, axis=1, keepdims=True).astype(jnp.int32)

    ii = jax.lax.broadcasted_iota(jnp.int32, (N_EXPERTS, N_EXPERTS), 0)
    jj = jax.lax.broadcasted_iota(jnp.int32, (N_EXPERTS, N_EXPERTS), 1)
    tri = (ii < jj).astype(jnp.float32)
    start_row = jax.lax.dot_general(nblk_row, tri, (((1,), (0,)), ((), ())),
                                    preferred_element_type=jnp.float32)
    off_row = start_row * TB

    d1 = jnp.sum(jnp.where(oh1, off_row + c_exc, 0.0), axis=1, keepdims=True)
    d2 = jnp.sum(jnp.where(oh2, off_row + c_exc, 0.0), axis=1, keepdims=True)
    dest_ref[...] = jnp.concatenate([d1, d2], axis=1).astype(jnp.int32)

    ones_col = jnp.ones((T, 1), jnp.float32)
    cnt_col = jax.lax.dot_general(count, ones_col, (((0,), (0,)), ((), ())),
                                  preferred_element_type=jnp.float32)
    nblk_col = jnp.floor((cnt_col + (TB - 1)) / TB)
    low = (jj < ii).astype(jnp.float32)
    start_col = jax.lax.dot_general(low, nblk_col, (((1,), (0,)), ((), ())),
                                    preferred_element_type=jnp.float32)
    ib = jax.lax.broadcasted_iota(jnp.int32, (N_EXPERTS, NB), 1)
    be = jnp.sum((start_col.astype(jnp.int32) <= ib).astype(jnp.int32),
                 axis=0, keepdims=True) - 1
    be_ref[...] = be


F_CHUNK = 256
NF = D_FF // F_CHUNK


def _gmm_kernel(be_ref, used_ref, xs_ref, wg_ref, wu_ref, wd_ref, y_ref):
    b = pl.program_id(0)
    f = pl.program_id(1)

    @pl.when(b < used_ref[0])
    def _():
        xb = xs_ref[...]
        g = _dot_t(xb, wg_ref[0])
        u = _dot_t(xb, wu_ref[0])
        h = (g * jax.nn.sigmoid(g)) * u
        yp = _dot_t(h, wd_ref[0])

        @pl.when(f == 0)
        def _init():
            y_ref[...] = yp

        @pl.when(f > 0)
        def _acc():
            y_ref[...] += yp


def _shared_kernel(x_ref, sg_ref, su_ref, sd_ref, o_ref):
    x = x_ref[...]
    g = _dot_t(x, sg_ref[...])
    u = _dot_t(x, su_ref[...])
    h = (g * jax.nn.sigmoid(g)) * u
    o_ref[...] = _dot_t(h, sd_ref[...])


def _sc_wid():
    return jax.lax.axis_index("s") * NC + jax.lax.axis_index("c")


def _sc_scatter_body(x_hbm, d0_hbm, d1_hbm, out_hbm, i0_v, i1_v, rows_v,
                     sem, sem2):
    wid = _sc_wid()
    base = wid * TPW
    pltpu.sync_copy(d0_hbm.at[wid], i0_v)
    pltpu.sync_copy(d1_hbm.at[wid], i1_v)
    pltpu.sync_copy(x_hbm.at[pl.ds(base, TPW)], rows_v)
    c0 = pltpu.async_copy(rows_v, out_hbm.at[i0_v], sem)
    c1 = pltpu.async_copy(rows_v, out_hbm.at[i1_v], sem2)
    c0.wait()
    c1.wait()


def _sc_scatter(x, d0, d1):
    mesh = plsc.VectorSubcoreMesh(core_axis_name="c", subcore_axis_name="s")
    k = functools.partial(
        pl.kernel, mesh=mesh,
        out_type=jax.ShapeDtypeStruct((L, HIDDEN), jnp.float32),
        scratch_types=[
            pltpu.VMEM((TPW,), jnp.int32),
            pltpu.VMEM((TPW,), jnp.int32),
            pltpu.VMEM((TPW, HIDDEN), jnp.float32),
            pltpu.SemaphoreType.DMA,
            pltpu.SemaphoreType.DMA,
        ],
    )(_sc_scatter_body)
    return k(x, d0, d1)


def _sc_combine_body(y_hbm, sh_hbm, d0_hbm, d1_hbm, w1_hbm, w2_hbm, out_hbm,
                     i0_v, i1_v, g0_v, g1_v, acc_v, w1_v, w2_v, sem, sem2):
    wid = _sc_wid()
    base = wid * TPW
    half_t = TPW // 2
    pltpu.sync_copy(d0_hbm.at[wid], i0_v)
    pltpu.sync_copy(d1_hbm.at[wid], i1_v)
    pltpu.sync_copy(w1_hbm.at[wid], w1_v)
    pltpu.sync_copy(w2_hbm.at[wid], w2_v)
    for half in range(2):
        c0 = pltpu.async_copy(y_hbm.at[i0_v.at[pl.ds(half * half_t, half_t)]],
                              g0_v, sem)
        c1 = pltpu.async_copy(y_hbm.at[i1_v.at[pl.ds(half * half_t, half_t)]],
                              g1_v, sem2)
        pltpu.sync_copy(sh_hbm.at[pl.ds(base + half * half_t, half_t)], acc_v)
        c0.wait()
        c1.wait()

        def tok_body(j, _):
            w1vec = w1_v[half * half_t + j, :]
            w2vec = w2_v[half * half_t + j, :]
            for k2 in range(HIDDEN // 16):
                sl = pl.ds(k2 * 16, 16)
                acc_v[j, sl] = (acc_v[j, sl] + w1vec * g0_v[j, sl]
                                + w2vec * g1_v[j, sl])
            return 0

        jax.lax.fori_loop(0, half_t, tok_body, 0, unroll=False)
        pltpu.sync_copy(acc_v, out_hbm.at[pl.ds(base + half * half_t, half_t)])


def _sc_combine(y_sorted, shared, d0, d1, w1r, w2r):
    mesh = plsc.VectorSubcoreMesh(core_axis_name="c", subcore_axis_name="s")
    half_t = TPW // 2
    k = functools.partial(
        pl.kernel, mesh=mesh,
        out_type=jax.ShapeDtypeStruct((T, HIDDEN), jnp.float32),
        scratch_types=[
            pltpu.VMEM((TPW,), jnp.int32),
            pltpu.VMEM((TPW,), jnp.int32),
            pltpu.VMEM((half_t, HIDDEN), jnp.float32),
            pltpu.VMEM((half_t, HIDDEN), jnp.float32),
            pltpu.VMEM((half_t, HIDDEN), jnp.float32),
            pltpu.VMEM((TPW, 16), jnp.float32),
            pltpu.VMEM((TPW, 16), jnp.float32),
            pltpu.SemaphoreType.DMA,
            pltpu.SemaphoreType.DMA,
        ],
    )(_sc_combine_body)
    return k(y_sorted, shared, d0, d1, w1r, w2r)


def kernel(hidden_states, gate_weight, e_score_correction_bias,
           w_gate, w_up, w_down, shared_gate, shared_up, shared_down):
    x = hidden_states
    bias2d = e_score_correction_bias.reshape(1, N_EXPERTS)

    dest, w1r, w2r, be, used = pl.pallas_call(
        _dispatch_kernel,
        grid=(1,),
        in_specs=[
            pl.BlockSpec((T, HIDDEN), lambda i: (0, 0)),
            pl.BlockSpec((N_EXPERTS, HIDDEN), lambda i: (0, 0)),
            pl.BlockSpec((1, N_EXPERTS), lambda i: (0, 0)),
        ],
        out_specs=(
            pl.BlockSpec((T, TOP_K), lambda i: (0, 0)),
            pl.BlockSpec((T, 16), lambda i: (0, 0)),
            pl.BlockSpec((T, 16), lambda i: (0, 0)),
            pl.BlockSpec((1, NB), lambda i: (0, 0)),
            pl.BlockSpec((1, 1), lambda i: (0, 0)),
        ),
        out_shape=(
            jax.ShapeDtypeStruct((T, TOP_K), jnp.int32),
            jax.ShapeDtypeStruct((T, 16), jnp.float32),
            jax.ShapeDtypeStruct((T, 16), jnp.float32),
            jax.ShapeDtypeStruct((1, NB), jnp.int32),
            jax.ShapeDtypeStruct((1, 1), jnp.int32),
        ),
    )(x, gate_weight, bias2d)

    d0 = dest[:, 0].reshape(NW, TPW)
    d1 = dest[:, 1].reshape(NW, TPW)
    w1r3 = w1r.reshape(NW, TPW, 16)
    w2r3 = w2r.reshape(NW, TPW, 16)

    x_sorted = _sc_scatter(x, d0, d1)

    y_sorted = pl.pallas_call(
        _gmm_kernel,
        grid_spec=pltpu.PrefetchScalarGridSpec(
            num_scalar_prefetch=2,
            grid=(NB, NF),
            in_specs=[
                pl.BlockSpec((TB, HIDDEN), lambda b, f, be, used: (b, 0)),
                pl.BlockSpec((1, F_CHUNK, HIDDEN),
                             lambda b, f, be, used: (be[b], f, 0)),
                pl.BlockSpec((1, F_CHUNK, HIDDEN),
                             lambda b, f, be, used: (be[b], f, 0)),
                pl.BlockSpec((1, HIDDEN, F_CHUNK),
                             lambda b, f, be, used: (be[b], 0, f)),
            ],
            out_specs=pl.BlockSpec((TB, HIDDEN),
                                   lambda b, f, be, used: (b, 0)),
        ),
        out_shape=jax.ShapeDtypeStruct((L, HIDDEN), jnp.float32),
        compiler_params=pltpu.CompilerParams(
            dimension_semantics=("arbitrary", "arbitrary")),
    )(be.reshape(NB), used.reshape(1), x_sorted, w_gate, w_up, w_down)

    STB = 512
    shared = pl.pallas_call(
        _shared_kernel,
        grid=(T // STB,),
        in_specs=[
            pl.BlockSpec((STB, HIDDEN), lambda i: (i, 0)),
            pl.BlockSpec((D_FF_SHARED, HIDDEN), lambda i: (0, 0)),
            pl.BlockSpec((D_FF_SHARED, HIDDEN), lambda i: (0, 0)),
            pl.BlockSpec((HIDDEN, D_FF_SHARED), lambda i: (0, 0)),
        ],
        out_specs=pl.BlockSpec((STB, HIDDEN), lambda i: (i, 0)),
        out_shape=jax.ShapeDtypeStruct((T, HIDDEN), jnp.float32),
        compiler_params=pltpu.CompilerParams(
            dimension_semantics=("parallel",)),
    )(x, shared_gate, shared_up, shared_down)

    return _sc_combine(y_sorted, shared, d0, d1, w1r3, w2r3)


# speedup vs baseline: 1.3176x; 1.3176x over previous
import functools

import jax
import jax.numpy as jnp
from jax.experimental import pallas as pl
from jax.experimental.pallas import tpu as pltpu
from jax.experimental.pallas import tpu_sc as plsc

T = 2048
HIDDEN = 1024
N_EXPERTS = 8
TOP_K = 2
D_FF = 768
D_FF_SHARED = 1536

TB = 256
L = T * TOP_K + N_EXPERTS * TB
NB = L // TB
NC = 2
NS = 16
NW = NC * NS
TPW = T // NW


def _dot_t(a, b):
    return jax.lax.dot_general(a.astype(jnp.bfloat16), b.astype(jnp.bfloat16),
                               (((1,), (1,)), ((), ())),
                               preferred_element_type=jnp.float32)


def _shift_down(a, s):
    return jnp.concatenate([jnp.zeros((s, a.shape[1]), a.dtype), a[:-s]],
                           axis=0)


def _dispatch_kernel(x_ref, gw_ref, bias_ref,
                     dest_ref, w1_ref, w2_ref, be_ref, used_ref):
    x = x_ref[...]
    logits = jax.lax.dot_general(x, gw_ref[...], (((1,), (1,)), ((), ())),
                                 preferred_element_type=jnp.float32)
    scores = jax.nn.sigmoid(logits)
    biased = scores + bias_ref[...]
    eiota = jax.lax.broadcasted_iota(jnp.int32, (T, N_EXPERTS), 1)
    m1 = jnp.max(biased, axis=1, keepdims=True)
    idx1 = jnp.min(jnp.where(biased == m1, eiota, N_EXPERTS),
                   axis=1, keepdims=True)
    oh1 = eiota == idx1
    b2 = jnp.where(oh1, -jnp.inf, biased)
    m2 = jnp.max(b2, axis=1, keepdims=True)
    idx2 = jnp.min(jnp.where(b2 == m2, eiota, N_EXPERTS),
                   axis=1, keepdims=True)
    oh2 = eiota == idx2
    w1 = jnp.sum(jnp.where(oh1, scores, 0.0), axis=1, keepdims=True)
    w2 = jnp.sum(jnp.where(oh2, scores, 0.0), axis=1, keepdims=True)
    den = w1 + w2 + 1e-20
    w1_ref[...] = jnp.broadcast_to(w1 / den, (T, 16))
    w2_ref[...] = jnp.broadcast_to(w2 / den, (T, 16))

    count = (oh1 | oh2).astype(jnp.float32)
    c = count
    s = 1
    while s < T:
        c = c + _shift_down(c, s)
        s *= 2
    c_exc = c - count
    cnt_row = c[T - 1:T, :]
    nblk_row = jnp.floor((cnt_row + (TB - 1)) / TB)
    used_ref[...] = jnp.sum(nblk_row, axis=1, keepdims=True).astype(jnp.int32)

    ii = jax.lax.broadcasted_iota(jnp.int32, (N_EXPERTS, N_EXPERTS), 0)
    jj = jax.lax.broadcasted_iota(jnp.int32, (N_EXPERTS, N_EXPERTS), 1)
    tri = (ii < jj).astype(jnp.float32)
    start_row = jax.lax.dot_general(nblk_row, tri, (((1,), (0,)), ((), ())),
                                    preferred_element_type=jnp.float32)
    off_row = start_row * TB

    d1 = jnp.sum(jnp.where(oh1, off_row + c_exc, 0.0), axis=1, keepdims=True)
    d2 = jnp.sum(jnp.where(oh2, off_row + c_exc, 0.0), axis=1, keepdims=True)
    dest_ref[...] = jnp.concatenate([d1, d2], axis=1).astype(jnp.int32)

    ones_col = jnp.ones((T, 1), jnp.float32)
    cnt_col = jax.lax.dot_general(count, ones_col, (((0,), (0,)), ((), ())),
                                  preferred_element_type=jnp.float32)
    nblk_col = jnp.floor((cnt_col + (TB - 1)) / TB)
    low = (jj < ii).astype(jnp.float32)
    start_col = jax.lax.dot_general(low, nblk_col, (((1,), (0,)), ((), ())),
                                    preferred_element_type=jnp.float32)
    ib = jax.lax.broadcasted_iota(jnp.int32, (N_EXPERTS, NB), 1)
    be = jnp.sum((start_col.astype(jnp.int32) <= ib).astype(jnp.int32),
                 axis=0, keepdims=True) - 1
    be_ref[...] = be


def _gmm_kernel(be_ref, used_ref, xs_ref, wg_ref, wu_ref, wd_ref, sh_ref,
                y_ref):
    b = pl.program_id(0)

    @pl.when(b < used_ref[0])
    def _():
        xb = xs_ref[...]
        g = _dot_t(xb, wg_ref[0])
        u = _dot_t(xb, wu_ref[0])
        h = (g * jax.nn.sigmoid(g)) * u
        y_ref[...] = _dot_t(h, wd_ref[0]) + 0.0 * sh_ref[0, 0]


def _shared_kernel(x_ref, sg_ref, su_ref, sd_ref, o_ref):
    x = x_ref[...]
    g = _dot_t(x, sg_ref[...])
    u = _dot_t(x, su_ref[...])
    h = (g * jax.nn.sigmoid(g)) * u
    o_ref[...] = _dot_t(h, sd_ref[...])


def _sc_wid():
    return jax.lax.axis_index("s") * NC + jax.lax.axis_index("c")


def _sc_scatter_body(x_hbm, d0_hbm, d1_hbm, out_hbm, i0_v, i1_v, rows_v,
                     sem, sem2):
    wid = _sc_wid()
    base = wid * TPW
    pltpu.sync_copy(d0_hbm.at[wid], i0_v)
    pltpu.sync_copy(d1_hbm.at[wid], i1_v)
    pltpu.sync_copy(x_hbm.at[pl.ds(base, TPW)], rows_v)
    c0 = pltpu.async_copy(rows_v, out_hbm.at[i0_v], sem)
    c1 = pltpu.async_copy(rows_v, out_hbm.at[i1_v], sem2)
    c0.wait()
    c1.wait()


def _sc_scatter(x, d0, d1):
    mesh = plsc.VectorSubcoreMesh(core_axis_name="c", subcore_axis_name="s")
    k = functools.partial(
        pl.kernel, mesh=mesh,
        out_type=jax.ShapeDtypeStruct((L, HIDDEN), jnp.float32),
        scratch_types=[
            pltpu.VMEM((TPW,), jnp.int32),
            pltpu.VMEM((TPW,), jnp.int32),
            pltpu.VMEM((TPW, HIDDEN), jnp.float32),
            pltpu.SemaphoreType.DMA,
            pltpu.SemaphoreType.DMA,
        ],
    )(_sc_scatter_body)
    return k(x, d0, d1)


def _sc_combine_body(y_hbm, sh_hbm, d0_hbm, d1_hbm, w1_hbm, w2_hbm, out_hbm,
                     i0_v, i1_v, g0_v, g1_v, acc_v, w1_v, w2_v, sem, sem2):
    wid = _sc_wid()
    base = wid * TPW
    half_t = TPW // 2
    pltpu.sync_copy(d0_hbm.at[wid], i0_v)
    pltpu.sync_copy(d1_hbm.at[wid], i1_v)
    pltpu.sync_copy(w1_hbm.at[wid], w1_v)
    pltpu.sync_copy(w2_hbm.at[wid], w2_v)
    for half in range(2):
        c0 = pltpu.async_copy(y_hbm.at[i0_v.at[pl.ds(half * half_t, half_t)]],
                              g0_v, sem)
        c1 = pltpu.async_copy(y_hbm.at[i1_v.at[pl.ds(half * half_t, half_t)]],
                              g1_v, sem2)
        pltpu.sync_copy(sh_hbm.at[pl.ds(base + half * half_t, half_t)], acc_v)
        c0.wait()
        c1.wait()

        def tok_body(j, _):
            w1vec = w1_v[half * half_t + j, :]
            w2vec = w2_v[half * half_t + j, :]
            for k2 in range(HIDDEN // 16):
                sl = pl.ds(k2 * 16, 16)
                acc_v[j, sl] = (acc_v[j, sl] + w1vec * g0_v[j, sl]
                                + w2vec * g1_v[j, sl])
            return 0

        jax.lax.fori_loop(0, half_t, tok_body, 0, unroll=False)
        pltpu.sync_copy(acc_v, out_hbm.at[pl.ds(base + half * half_t, half_t)])


def _sc_combine(y_sorted, shared, d0, d1, w1r, w2r):
    mesh = plsc.VectorSubcoreMesh(core_axis_name="c", subcore_axis_name="s")
    half_t = TPW // 2
    k = functools.partial(
        pl.kernel, mesh=mesh,
        out_type=jax.ShapeDtypeStruct((T, HIDDEN), jnp.float32),
        scratch_types=[
            pltpu.VMEM((TPW,), jnp.int32),
            pltpu.VMEM((TPW,), jnp.int32),
            pltpu.VMEM((half_t, HIDDEN), jnp.float32),
            pltpu.VMEM((half_t, HIDDEN), jnp.float32),
            pltpu.VMEM((half_t, HIDDEN), jnp.float32),
            pltpu.VMEM((TPW, 16), jnp.float32),
            pltpu.VMEM((TPW, 16), jnp.float32),
            pltpu.SemaphoreType.DMA,
            pltpu.SemaphoreType.DMA,
        ],
    )(_sc_combine_body)
    return k(y_sorted, shared, d0, d1, w1r, w2r)


def kernel(hidden_states, gate_weight, e_score_correction_bias,
           w_gate, w_up, w_down, shared_gate, shared_up, shared_down):
    x = hidden_states
    bias2d = e_score_correction_bias.reshape(1, N_EXPERTS)

    dest, w1r, w2r, be, used = pl.pallas_call(
        _dispatch_kernel,
        grid=(1,),
        in_specs=[
            pl.BlockSpec((T, HIDDEN), lambda i: (0, 0)),
            pl.BlockSpec((N_EXPERTS, HIDDEN), lambda i: (0, 0)),
            pl.BlockSpec((1, N_EXPERTS), lambda i: (0, 0)),
        ],
        out_specs=(
            pl.BlockSpec((T, TOP_K), lambda i: (0, 0)),
            pl.BlockSpec((T, 16), lambda i: (0, 0)),
            pl.BlockSpec((T, 16), lambda i: (0, 0)),
            pl.BlockSpec((1, NB), lambda i: (0, 0)),
            pl.BlockSpec((1, 1), lambda i: (0, 0)),
        ),
        out_shape=(
            jax.ShapeDtypeStruct((T, TOP_K), jnp.int32),
            jax.ShapeDtypeStruct((T, 16), jnp.float32),
            jax.ShapeDtypeStruct((T, 16), jnp.float32),
            jax.ShapeDtypeStruct((1, NB), jnp.int32),
            jax.ShapeDtypeStruct((1, 1), jnp.int32),
        ),
    )(x, gate_weight, bias2d)

    d0 = dest[:, 0].reshape(NW, TPW)
    d1 = dest[:, 1].reshape(NW, TPW)
    w1r3 = w1r.reshape(NW, TPW, 16)
    w2r3 = w2r.reshape(NW, TPW, 16)

    x_sorted = _sc_scatter(x, d0, d1)

    STB = 512
    shared = pl.pallas_call(
        _shared_kernel,
        grid=(T // STB,),
        in_specs=[
            pl.BlockSpec((STB, HIDDEN), lambda i: (i, 0)),
            pl.BlockSpec((D_FF_SHARED, HIDDEN), lambda i: (0, 0)),
            pl.BlockSpec((D_FF_SHARED, HIDDEN), lambda i: (0, 0)),
            pl.BlockSpec((HIDDEN, D_FF_SHARED), lambda i: (0, 0)),
        ],
        out_specs=pl.BlockSpec((STB, HIDDEN), lambda i: (i, 0)),
        out_shape=jax.ShapeDtypeStruct((T, HIDDEN), jnp.float32),
        compiler_params=pltpu.CompilerParams(
            dimension_semantics=("parallel",)),
    )(x, shared_gate, shared_up, shared_down)

    y_sorted = pl.pallas_call(
        _gmm_kernel,
        grid_spec=pltpu.PrefetchScalarGridSpec(
            num_scalar_prefetch=2,
            grid=(NB,),
            in_specs=[
                pl.BlockSpec((TB, HIDDEN), lambda b, be, used: (b, 0)),
                pl.BlockSpec((1, D_FF, HIDDEN),
                             lambda b, be, used: (be[b], 0, 0)),
                pl.BlockSpec((1, D_FF, HIDDEN),
                             lambda b, be, used: (be[b], 0, 0)),
                pl.BlockSpec((1, HIDDEN, D_FF),
                             lambda b, be, used: (be[b], 0, 0)),
                pl.BlockSpec((8, 128), lambda b, be, used: (0, 0)),
            ],
            out_specs=pl.BlockSpec((TB, HIDDEN), lambda b, be, used: (b, 0)),
        ),
        out_shape=jax.ShapeDtypeStruct((L, HIDDEN), jnp.float32),
        compiler_params=pltpu.CompilerParams(
            dimension_semantics=("arbitrary",)),
    )(be.reshape(NB), used.reshape(1), x_sorted, w_gate, w_up, w_down, shared)

    return _sc_combine(y_sorted, shared, d0, d1, w1r3, w2r3)


# speedup vs baseline: 1.3388x; 1.0160x over previous
import functools

import jax
import jax.numpy as jnp
from jax.experimental import pallas as pl
from jax.experimental.pallas import tpu as pltpu
from jax.experimental.pallas import tpu_sc as plsc

T = 2048
HIDDEN = 1024
N_EXPERTS = 8
TOP_K = 2
D_FF = 768
D_FF_SHARED = 1536

TB = 512
L = T * TOP_K + N_EXPERTS * TB
NB = L // TB
NC = 2
NS = 16
NW = NC * NS
TPW = T // NW


def _dot_t(a, b):
    return jax.lax.dot_general(a.astype(jnp.bfloat16), b.astype(jnp.bfloat16),
                               (((1,), (1,)), ((), ())),
                               preferred_element_type=jnp.float32)


def _shift_down(a, s):
    return jnp.concatenate([jnp.zeros((s, a.shape[1]), a.dtype), a[:-s]],
                           axis=0)


def _dispatch_kernel(x_ref, gw_ref, bias_ref,
                     dest_ref, w1_ref, w2_ref, be_ref, used_ref):
    x = x_ref[...]
    logits = jax.lax.dot_general(x, gw_ref[...], (((1,), (1,)), ((), ())),
                                 preferred_element_type=jnp.float32)
    scores = jax.nn.sigmoid(logits)
    biased = scores + bias_ref[...]
    eiota = jax.lax.broadcasted_iota(jnp.int32, (T, N_EXPERTS), 1)
    m1 = jnp.max(biased, axis=1, keepdims=True)
    idx1 = jnp.min(jnp.where(biased == m1, eiota, N_EXPERTS),
                   axis=1, keepdims=True)
    oh1 = eiota == idx1
    b2 = jnp.where(oh1, -jnp.inf, biased)
    m2 = jnp.max(b2, axis=1, keepdims=True)
    idx2 = jnp.min(jnp.where(b2 == m2, eiota, N_EXPERTS),
                   axis=1, keepdims=True)
    oh2 = eiota == idx2
    w1 = jnp.sum(jnp.where(oh1, scores, 0.0), axis=1, keepdims=True)
    w2 = jnp.sum(jnp.where(oh2, scores, 0.0), axis=1, keepdims=True)
    den = w1 + w2 + 1e-20
    w1_ref[...] = jnp.broadcast_to(w1 / den, (T, 16))
    w2_ref[...] = jnp.broadcast_to(w2 / den, (T, 16))

    count = (oh1 | oh2).astype(jnp.float32)
    c = count
    s = 1
    while s < T:
        c = c + _shift_down(c, s)
        s *= 2
    c_exc = c - count
    cnt_row = c[T - 1:T, :]
    nblk_row = jnp.floor((cnt_row + (TB - 1)) / TB)
    used_ref[...] = jnp.sum(nblk_row, axis=1, keepdims=True).astype(jnp.int32)

    ii = jax.lax.broadcasted_iota(jnp.int32, (N_EXPERTS, N_EXPERTS), 0)
    jj = jax.lax.broadcasted_iota(jnp.int32, (N_EXPERTS, N_EXPERTS), 1)
    tri = (ii < jj).astype(jnp.float32)
    start_row = jax.lax.dot_general(nblk_row, tri, (((1,), (0,)), ((), ())),
                                    preferred_element_type=jnp.float32)
    off_row = start_row * TB

    d1 = jnp.sum(jnp.where(oh1, off_row + c_exc, 0.0), axis=1, keepdims=True)
    d2 = jnp.sum(jnp.where(oh2, off_row + c_exc, 0.0), axis=1, keepdims=True)
    dest_ref[...] = jnp.concatenate([d1, d2], axis=1).astype(jnp.int32)

    ones_col = jnp.ones((T, 1), jnp.float32)
    cnt_col = jax.lax.dot_general(count, ones_col, (((0,), (0,)), ((), ())),
                                  preferred_element_type=jnp.float32)
    nblk_col = jnp.floor((cnt_col + (TB - 1)) / TB)
    low = (jj < ii).astype(jnp.float32)
    start_col = jax.lax.dot_general(low, nblk_col, (((1,), (0,)), ((), ())),
                                    preferred_element_type=jnp.float32)
    ib = jax.lax.broadcasted_iota(jnp.int32, (N_EXPERTS, NB), 1)
    be = jnp.sum((start_col.astype(jnp.int32) <= ib).astype(jnp.int32),
                 axis=0, keepdims=True) - 1
    be_ref[...] = be


def _gmm_kernel(be_ref, used_ref, xs_ref, wg_ref, wu_ref, wd_ref, sh_ref,
                y_ref):
    b = pl.program_id(0)

    @pl.when(b < used_ref[0])
    def _():
        xb = xs_ref[...]
        g = _dot_t(xb, wg_ref[0])
        u = _dot_t(xb, wu_ref[0])
        h = (g * jax.nn.sigmoid(g)) * u
        y_ref[...] = _dot_t(h, wd_ref[0]) + 0.0 * sh_ref[0, 0]


def _shared_kernel(x_ref, sg_ref, su_ref, sd_ref, o_ref):
    x = x_ref[...]
    g = _dot_t(x, sg_ref[...])
    u = _dot_t(x, su_ref[...])
    h = (g * jax.nn.sigmoid(g)) * u
    o_ref[...] = _dot_t(h, sd_ref[...])


def _sc_wid():
    return jax.lax.axis_index("s") * NC + jax.lax.axis_index("c")


def _sc_scatter_body(x_hbm, d0_hbm, d1_hbm, out_hbm, i0_v, i1_v, rows_v,
                     sem, sem2):
    wid = _sc_wid()
    base = wid * TPW
    pltpu.sync_copy(d0_hbm.at[wid], i0_v)
    pltpu.sync_copy(d1_hbm.at[wid], i1_v)
    pltpu.sync_copy(x_hbm.at[pl.ds(base, TPW)], rows_v)
    c0 = pltpu.async_copy(rows_v, out_hbm.at[i0_v], sem)
    c1 = pltpu.async_copy(rows_v, out_hbm.at[i1_v], sem2)
    c0.wait()
    c1.wait()


def _sc_scatter(x, d0, d1):
    mesh = plsc.VectorSubcoreMesh(core_axis_name="c", subcore_axis_name="s")
    k = functools.partial(
        pl.kernel, mesh=mesh,
        out_type=jax.ShapeDtypeStruct((L, HIDDEN), jnp.float32),
        scratch_types=[
            pltpu.VMEM((TPW,), jnp.int32),
            pltpu.VMEM((TPW,), jnp.int32),
            pltpu.VMEM((TPW, HIDDEN), jnp.float32),
            pltpu.SemaphoreType.DMA,
            pltpu.SemaphoreType.DMA,
        ],
    )(_sc_scatter_body)
    return k(x, d0, d1)


def _sc_combine_body(y_hbm, sh_hbm, d0_hbm, d1_hbm, w1_hbm, w2_hbm, out_hbm,
                     i0_v, i1_v, g0_v, g1_v, acc_v, w1_v, w2_v, sem, sem2):
    wid = _sc_wid()
    base = wid * TPW
    half_t = TPW // 2
    pltpu.sync_copy(d0_hbm.at[wid], i0_v)
    pltpu.sync_copy(d1_hbm.at[wid], i1_v)
    pltpu.sync_copy(w1_hbm.at[wid], w1_v)
    pltpu.sync_copy(w2_hbm.at[wid], w2_v)
    for half in range(2):
        c0 = pltpu.async_copy(y_hbm.at[i0_v.at[pl.ds(half * half_t, half_t)]],
                              g0_v, sem)
        c1 = pltpu.async_copy(y_hbm.at[i1_v.at[pl.ds(half * half_t, half_t)]],
                              g1_v, sem2)
        pltpu.sync_copy(sh_hbm.at[pl.ds(base + half * half_t, half_t)], acc_v)
        c0.wait()
        c1.wait()

        def tok_body(j, _):
            w1vec = w1_v[half * half_t + j, :]
            w2vec = w2_v[half * half_t + j, :]
            for k2 in range(HIDDEN // 16):
                sl = pl.ds(k2 * 16, 16)
                acc_v[j, sl] = (acc_v[j, sl] + w1vec * g0_v[j, sl]
                                + w2vec * g1_v[j, sl])
            return 0

        jax.lax.fori_loop(0, half_t, tok_body, 0, unroll=False)
        pltpu.sync_copy(acc_v, out_hbm.at[pl.ds(base + half * half_t, half_t)])


def _sc_combine(y_sorted, shared, d0, d1, w1r, w2r):
    mesh = plsc.VectorSubcoreMesh(core_axis_name="c", subcore_axis_name="s")
    half_t = TPW // 2
    k = functools.partial(
        pl.kernel, mesh=mesh,
        out_type=jax.ShapeDtypeStruct((T, HIDDEN), jnp.float32),
        scratch_types=[
            pltpu.VMEM((TPW,), jnp.int32),
            pltpu.VMEM((TPW,), jnp.int32),
            pltpu.VMEM((half_t, HIDDEN), jnp.float32),
            pltpu.VMEM((half_t, HIDDEN), jnp.float32),
            pltpu.VMEM((half_t, HIDDEN), jnp.float32),
            pltpu.VMEM((TPW, 16), jnp.float32),
            pltpu.VMEM((TPW, 16), jnp.float32),
            pltpu.SemaphoreType.DMA,
            pltpu.SemaphoreType.DMA,
        ],
    )(_sc_combine_body)
    return k(y_sorted, shared, d0, d1, w1r, w2r)


def kernel(hidden_states, gate_weight, e_score_correction_bias,
           w_gate, w_up, w_down, shared_gate, shared_up, shared_down):
    x = hidden_states
    bias2d = e_score_correction_bias.reshape(1, N_EXPERTS)

    dest, w1r, w2r, be, used = pl.pallas_call(
        _dispatch_kernel,
        grid=(1,),
        in_specs=[
            pl.BlockSpec((T, HIDDEN), lambda i: (0, 0)),
            pl.BlockSpec((N_EXPERTS, HIDDEN), lambda i: (0, 0)),
            pl.BlockSpec((1, N_EXPERTS), lambda i: (0, 0)),
        ],
        out_specs=(
            pl.BlockSpec((T, TOP_K), lambda i: (0, 0)),
            pl.BlockSpec((T, 16), lambda i: (0, 0)),
            pl.BlockSpec((T, 16), lambda i: (0, 0)),
            pl.BlockSpec((1, NB), lambda i: (0, 0)),
            pl.BlockSpec((1, 1), lambda i: (0, 0)),
        ),
        out_shape=(
            jax.ShapeDtypeStruct((T, TOP_K), jnp.int32),
            jax.ShapeDtypeStruct((T, 16), jnp.float32),
            jax.ShapeDtypeStruct((T, 16), jnp.float32),
            jax.ShapeDtypeStruct((1, NB), jnp.int32),
            jax.ShapeDtypeStruct((1, 1), jnp.int32),
        ),
    )(x, gate_weight, bias2d)

    d0 = dest[:, 0].reshape(NW, TPW)
    d1 = dest[:, 1].reshape(NW, TPW)
    w1r3 = w1r.reshape(NW, TPW, 16)
    w2r3 = w2r.reshape(NW, TPW, 16)

    x_sorted = _sc_scatter(x, d0, d1)

    STB = 512
    shared = pl.pallas_call(
        _shared_kernel,
        grid=(T // STB,),
        in_specs=[
            pl.BlockSpec((STB, HIDDEN), lambda i: (i, 0)),
            pl.BlockSpec((D_FF_SHARED, HIDDEN), lambda i: (0, 0)),
            pl.BlockSpec((D_FF_SHARED, HIDDEN), lambda i: (0, 0)),
            pl.BlockSpec((HIDDEN, D_FF_SHARED), lambda i: (0, 0)),
        ],
        out_specs=pl.BlockSpec((STB, HIDDEN), lambda i: (i, 0)),
        out_shape=jax.ShapeDtypeStruct((T, HIDDEN), jnp.float32),
        compiler_params=pltpu.CompilerParams(
            dimension_semantics=("parallel",)),
    )(x, shared_gate, shared_up, shared_down)

    y_sorted = pl.pallas_call(
        _gmm_kernel,
        grid_spec=pltpu.PrefetchScalarGridSpec(
            num_scalar_prefetch=2,
            grid=(NB,),
            in_specs=[
                pl.BlockSpec((TB, HIDDEN), lambda b, be, used: (b, 0)),
                pl.BlockSpec((1, D_FF, HIDDEN),
                             lambda b, be, used: (be[b], 0, 0)),
                pl.BlockSpec((1, D_FF, HIDDEN),
                             lambda b, be, used: (be[b], 0, 0)),
                pl.BlockSpec((1, HIDDEN, D_FF),
                             lambda b, be, used: (be[b], 0, 0)),
                pl.BlockSpec((8, 128), lambda b, be, used: (0, 0)),
            ],
            out_specs=pl.BlockSpec((TB, HIDDEN), lambda b, be, used: (b, 0)),
        ),
        out_shape=jax.ShapeDtypeStruct((L, HIDDEN), jnp.float32),
        compiler_params=pltpu.CompilerParams(
            dimension_semantics=("arbitrary",)),
    )(be.reshape(NB), used.reshape(1), x_sorted, w_gate, w_up, w_down, shared)

    return _sc_combine(y_sorted, shared, d0, d1, w1r3, w2r3)
